# baseline (speedup 1.0000x reference)
import functools

import jax
import jax.numpy as jnp
from jax import lax
from jax.experimental import pallas as pl
from jax.experimental.pallas import tpu as pltpu
from jax.experimental.pallas import tpu_sc as plsc

N = 10000
E = 320000
D = 128
K = 5000
NP = 10240
NROWS = 80

NC = 2
NS = 16
NW = NC * NS
EPW = E // NW
CHUNK = 128
CH_PROC = 80
CH_ALL = 82
SLOTS = CH_ALL * CHUNK
RPS = NP // NS
SW = 8

_SC_MESH = dict(core_axis_name="c", subcore_axis_name="s")


def _sc_degrees(src, dst, ones_c, zeros_col):
    @functools.partial(
        pl.kernel,
        out_type=(
            jax.ShapeDtypeStruct((NC, NP, SW), jnp.float32),
            jax.ShapeDtypeStruct((NC, NP, SW), jnp.float32),
        ),
        mesh=plsc.VectorSubcoreMesh(**_SC_MESH),
        scratch_types=[
            pltpu.VMEM((CHUNK,), jnp.int32),
            pltpu.VMEM((CHUNK, SW), jnp.float32),
            pltpu.VMEM_SHARED((NP, SW), jnp.float32),
            pltpu.VMEM_SHARED((NP, SW), jnp.float32),
        ],
        compiler_params=pltpu.CompilerParams(use_tc_tiling_on_sc=False),
    )
    def body(src_hbm, dst_hbm, ones_hbm, zcol_hbm, dego_hbm, degi_hbm,
             idxv, onesv, acc_o, acc_i):
        cid = lax.axis_index("c")
        sid = lax.axis_index("s")
        wid = cid * NS + sid
        pltpu.sync_copy(ones_hbm, onesv)
        sl = pl.ds(sid * RPS, RPS)
        pltpu.sync_copy(zcol_hbm.at[sl], acc_o.at[sl])
        pltpu.sync_copy(zcol_hbm.at[sl], acc_i.at[sl])
        plsc.subcore_barrier()
        base0 = wid * SLOTS

        def step(i, carry):
            base = base0 + i * CHUNK
            pltpu.sync_copy(src_hbm.at[pl.ds(base, CHUNK)], idxv)
            pltpu.sync_copy(onesv, acc_o.at[idxv], add=True)
            pltpu.sync_copy(dst_hbm.at[pl.ds(base, CHUNK)], idxv)
            pltpu.sync_copy(onesv, acc_i.at[idxv], add=True)
            return carry

        lax.fori_loop(0, CH_ALL, step, 0)
        plsc.subcore_barrier()
        pltpu.sync_copy(acc_o.at[sl], dego_hbm.at[cid, sl])
        pltpu.sync_copy(acc_i.at[sl], degi_hbm.at[cid, sl])

    return body(src, dst, ones_c, zeros_col)


def _sc_segment_sum(h, src, dst, zeros):
    @functools.partial(
        pl.kernel,
        out_type=jax.ShapeDtypeStruct((NC, NP, D), jnp.float32),
        mesh=plsc.VectorSubcoreMesh(**_SC_MESH),
        scratch_types=[
            pltpu.VMEM((CHUNK,), jnp.int32),
            pltpu.VMEM((CHUNK,), jnp.int32),
            pltpu.VMEM((CHUNK,), jnp.int32),
            pltpu.VMEM((CHUNK,), jnp.int32),
            pltpu.VMEM((CHUNK, D), jnp.float32),
            pltpu.VMEM((CHUNK, D), jnp.float32),
            pltpu.VMEM_SHARED((NP, D), jnp.float32),
            pltpu.SemaphoreType.DMA,
            pltpu.SemaphoreType.DMA,
        ],
    )
    def body(h_hbm, src_hbm, dst_hbm, z_hbm, out_hbm,
             srcv0, dstv0, srcv1, dstv1, rows0, rows1, acc, sem0, sem1):
        cid = lax.axis_index("c")
        sid = lax.axis_index("s")
        wid = cid * NS + sid
        sl = pl.ds(sid * RPS, RPS)
        pltpu.sync_copy(z_hbm.at[sl], acc.at[sl])
        plsc.subcore_barrier()
        base0 = wid * SLOTS

        def fetch(c, srcv, dstv, rows, sem):
            base = base0 + c * CHUNK
            pltpu.sync_copy(src_hbm.at[pl.ds(base, CHUNK)], srcv)
            pltpu.sync_copy(dst_hbm.at[pl.ds(base, CHUNK)], dstv)
            pltpu.async_copy(h_hbm.at[srcv], rows, sem)

        fetch(0, srcv0, dstv0, rows0, sem0)
        fetch(1, srcv1, dstv1, rows1, sem1)

        def step(i, carry):
            pltpu.make_async_copy(h_hbm.at[srcv0], rows0, sem0).wait()
            pltpu.sync_copy(rows0, acc.at[dstv0], add=True)
            fetch(2 * i + 2, srcv0, dstv0, rows0, sem0)
            pltpu.make_async_copy(h_hbm.at[srcv1], rows1, sem1).wait()
            pltpu.sync_copy(rows1, acc.at[dstv1], add=True)
            fetch(2 * i + 3, srcv1, dstv1, rows1, sem1)
            return carry

        lax.fori_loop(0, CH_PROC // 2, step, 0)
        pltpu.make_async_copy(h_hbm.at[srcv0], rows0, sem0).wait()
        pltpu.make_async_copy(h_hbm.at[srcv1], rows1, sem1).wait()
        plsc.subcore_barrier()
        pltpu.sync_copy(acc.at[sl], out_hbm.at[cid, sl])

    return body(h, src, dst, zeros)


def _sc_segment_sum_scalar(s, src, dst, zeros_col):
    @functools.partial(
        pl.kernel,
        out_type=jax.ShapeDtypeStruct((NC, NP, SW), jnp.float32),
        mesh=plsc.VectorSubcoreMesh(**_SC_MESH),
        scratch_types=[
            pltpu.VMEM((CHUNK,), jnp.int32),
            pltpu.VMEM((CHUNK,), jnp.int32),
            pltpu.VMEM((CHUNK, SW), jnp.float32),
            pltpu.VMEM_SHARED((NP, SW), jnp.float32),
            pltpu.SemaphoreType.DMA,
        ],
        compiler_params=pltpu.CompilerParams(use_tc_tiling_on_sc=False),
    )
    def body(s_hbm, src_hbm, dst_hbm, z_hbm, out_hbm, srcv, dstv, vals, acc, sem):
        cid = lax.axis_index("c")
        sid = lax.axis_index("s")
        wid = cid * NS + sid
        sl = pl.ds(sid * RPS, RPS)
        pltpu.sync_copy(z_hbm.at[sl], acc.at[sl])
        plsc.subcore_barrier()
        base0 = wid * SLOTS

        def step(i, carry):
            base = base0 + i * CHUNK
            pltpu.sync_copy(src_hbm.at[pl.ds(base, CHUNK)], srcv)
            pltpu.sync_copy(dst_hbm.at[pl.ds(base, CHUNK)], dstv)
            pltpu.async_copy(s_hbm.at[srcv], vals, sem).wait()
            pltpu.sync_copy(vals, acc.at[dstv], add=True)
            return carry

        lax.fori_loop(0, CH_ALL, step, 0)
        plsc.subcore_barrier()
        pltpu.sync_copy(acc.at[sl], out_hbm.at[cid, sl])

    return body(s, src, dst, zeros_col)


def _tc_prep(do0, do1, di0, di1):
    def body(a_ref, b_ref, c_ref, d_ref, ro_ref, ri_ref):
        ro_ref[...] = lax.rsqrt(jnp.maximum(a_ref[...] + b_ref[...], 1.0))
        ri_ref[...] = lax.rsqrt(jnp.maximum(c_ref[...] + d_ref[...], 1.0))

    return pl.pallas_call(
        body,
        out_shape=(
            jax.ShapeDtypeStruct((NROWS, 128), jnp.float32),
            jax.ShapeDtypeStruct((NROWS, 128), jnp.float32),
        ),
    )(do0, do1, di0, di1)


_BLK = 512
_NBLK = NP // _BLK


def _tc_scale_rows(x, r_col):
    def body(x_ref, r_ref, o_ref):
        o_ref[...] = x_ref[...] * r_ref[...]

    return pl.pallas_call(
        body,
        grid=(_NBLK,),
        in_specs=[
            pl.BlockSpec((_BLK, D), lambda i: (i, 0)),
            pl.BlockSpec((_BLK, 1), lambda i: (i, 0)),
        ],
        out_specs=pl.BlockSpec((_BLK, D), lambda i: (i, 0)),
        out_shape=jax.ShapeDtypeStruct((NP, D), jnp.float32),
    )(x, r_col)


def _tc_layer(m0, m1, r_in, r_out, W, b, Ws, s_in):
    def body(m0_ref, m1_ref, ri_ref, ro_ref, w_ref, b_ref, ws_ref, si_ref,
             f_ref, h_ref, s_ref):
        m = (m0_ref[...] + m1_ref[...]) * ri_ref[...]
        f = jnp.dot(m, w_ref[...], preferred_element_type=jnp.float32) + b_ref[...]
        h = f * ro_ref[...]
        f_ref[...] = f
        h_ref[...] = h
        s_ref[...] = si_ref[...] + jnp.dot(h, ws_ref[...],
                                           preferred_element_type=jnp.float32)

    return pl.pallas_call(
        body,
        grid=(_NBLK,),
        in_specs=[
            pl.BlockSpec((_BLK, D), lambda i: (i, 0)),
            pl.BlockSpec((_BLK, D), lambda i: (i, 0)),
            pl.BlockSpec((_BLK, 1), lambda i: (i, 0)),
            pl.BlockSpec((_BLK, 1), lambda i: (i, 0)),
            pl.BlockSpec((D, D), lambda i: (0, 0)),
            pl.BlockSpec((1, D), lambda i: (0, 0)),
            pl.BlockSpec((D, 1), lambda i: (0, 0)),
            pl.BlockSpec((_BLK, 1), lambda i: (i, 0)),
        ],
        out_specs=(
            pl.BlockSpec((_BLK, D), lambda i: (i, 0)),
            pl.BlockSpec((_BLK, D), lambda i: (i, 0)),
            pl.BlockSpec((_BLK, 1), lambda i: (i, 0)),
        ),
        out_shape=(
            jax.ShapeDtypeStruct((NP, D), jnp.float32),
            jax.ShapeDtypeStruct((NP, D), jnp.float32),
            jax.ShapeDtypeStruct((NP, 1), jnp.float32),
        ),
    )(m0, m1, r_in, r_out, W, b, Ws, s_in)


def _tc_select(sp0, sp1, ri_pad, score_b):
    def body(a_ref, b_ref, r_ref, sb_ref, w_ref, sel_ref):
        score = (a_ref[...] + b_ref[...]) * r_ref[...] + sb_ref[0, 0]
        flat = (lax.broadcasted_iota(jnp.int32, (NROWS, 128), 0) * 128
                + lax.broadcasted_iota(jnp.int32, (NROWS, 128), 1))
        valid = flat < N
        score = jnp.where(valid, score, -jnp.inf)
        u = lax.bitcast_convert_type(score, jnp.uint32)
        key = jnp.where(u >> 31 == jnp.uint32(1), ~u, u | jnp.uint32(0x80000000))

        def bstep(_, lohi):
            lo, hi = lohi
            mid = lo + ((hi - lo) >> 1)
            c = jnp.sum((key > mid).astype(jnp.int32))
            pred = c < K
            return (jnp.where(pred, lo, mid + 1), jnp.where(pred, mid, hi))

        lo, _ = lax.fori_loop(0, 32, bstep,
                              (jnp.uint32(0), jnp.uint32(0xFFFFFFFF)))
        tau = lo
        gt = key > tau
        eq = key == tau
        c_gt = jnp.sum(gt.astype(jnp.int32))
        need = (K - c_gt).astype(jnp.float32)
        eqf = eq.astype(jnp.float32)
        cj = (lax.broadcasted_iota(jnp.int32, (128, 128), 0)
              < lax.broadcasted_iota(jnp.int32, (128, 128), 1))
        in_row = jnp.dot(eqf, cj.astype(jnp.float32),
                         preferred_element_type=jnp.float32)
        rows_eq = jnp.sum(eqf, axis=1, keepdims=True)
        rq = (lax.broadcasted_iota(jnp.int32, (NROWS, NROWS), 1)
              < lax.broadcasted_iota(jnp.int32, (NROWS, NROWS), 0))
        pre_row = jnp.dot(rq.astype(jnp.float32), rows_eq,
                          preferred_element_type=jnp.float32)
        rank = pre_row + in_row
        sel = gt | (eq & (rank < need))
        w_ref[...] = jnp.where(sel, jnp.tanh(score), 0.0)
        sel_ref[...] = sel.astype(jnp.float32)

    return pl.pallas_call(
        body,
        out_shape=(
            jax.ShapeDtypeStruct((NROWS, 128), jnp.float32),
            jax.ShapeDtypeStruct((NROWS, 128), jnp.float32),
        ),
    )(sp0, sp1, ri_pad, score_b)


def _tc_head(f1, f2, f3, w_col, sel_col, l1W, l1b, l2W, l2b, l3W, l3b):
    def body(f1_ref, f2_ref, f3_ref, w_ref, sel_ref,
             l1w_ref, l1b_ref, l2w_ref, l2b_ref, l3w_ref, l3b_ref,
             logits_ref, feat_ref):
        w = w_ref[...]
        selected = sel_ref[...] > 0.0
        parts_avg = []
        parts_max = []
        for f_ref in (f1_ref, f2_ref, f3_ref):
            p = f_ref[...] * w
            parts_avg.append(jnp.sum(p, axis=0, keepdims=True) * (1.0 / K))
            parts_max.append(jnp.max(jnp.where(selected, p, -3.4e38),
                                     axis=0, keepdims=True))
        feat0 = jnp.concatenate(parts_avg + parts_max, axis=1)
        h1 = jnp.maximum(
            jnp.dot(feat0, l1w_ref[...], preferred_element_type=jnp.float32)
            + l1b_ref[...], 0.0)
        h2 = jnp.maximum(
            jnp.dot(h1, l2w_ref[...], preferred_element_type=jnp.float32)
            + l2b_ref[...], 0.0)
        z = jnp.dot(h2, l3w_ref[...], preferred_element_type=jnp.float32) \
            + l3b_ref[...]
        zm = z - jnp.max(z, axis=1, keepdims=True)
        logits_ref[...] = zm - jnp.log(jnp.sum(jnp.exp(zm), axis=1,
                                               keepdims=True))
        feat_ref[...] = h2

    return pl.pallas_call(
        body,
        out_shape=(
            jax.ShapeDtypeStruct((1, 10), jnp.float32),
            jax.ShapeDtypeStruct((1, D), jnp.float32),
        ),
    )(f1, f2, f3, w_col, sel_col, l1W, l1b, l2W, l2b, l3W, l3b)


def kernel(x, edge_index, conv_W0, conv_b0, conv_W1, conv_b1, conv_W2,
           conv_b2, score_W, score_b, lin1_W, lin1_b, lin2_W, lin2_b,
           lin3_W, lin3_b):
    def pad_edges(a):
        a = jnp.reshape(a.astype(jnp.int32), (NW, EPW))
        a = jnp.pad(a, ((0, 0), (0, SLOTS - EPW)), constant_values=N)
        return jnp.reshape(a, (-1,))

    src = pad_edges(edge_index[0])
    dst = pad_edges(edge_index[1])
    x_pad = jnp.pad(x, ((0, NP - N), (0, 0)))
    zeros = jnp.zeros((NP, D), jnp.float32)
    zeros_col = jnp.zeros((NP, 1), jnp.float32)
    zeros_sw = jnp.zeros((NP, SW), jnp.float32)
    ones_c = jnp.ones((CHUNK, SW), jnp.float32)

    dego, degi = _sc_degrees(src, dst, ones_c, zeros_sw)
    ro_pad, ri_pad = _tc_prep(
        jnp.reshape(dego[0, :, 0], (NROWS, 128)),
        jnp.reshape(dego[1, :, 0], (NROWS, 128)),
        jnp.reshape(degi[0, :, 0], (NROWS, 128)),
        jnp.reshape(degi[1, :, 0], (NROWS, 128)))
    r_out = jnp.reshape(ro_pad, (NP, 1))
    r_in = jnp.reshape(ri_pad, (NP, 1))

    h = _tc_scale_rows(x_pad, r_out)
    s = zeros_col
    feats = []
    for W, b, Ws in (
            (conv_W0, conv_b0, score_W[0:D]),
            (conv_W1, conv_b1, score_W[D:2 * D]),
            (conv_W2, conv_b2, score_W[2 * D:3 * D])):
        m = _sc_segment_sum(h, src, dst, zeros)
        f, h, s = _tc_layer(m[0], m[1], r_in, r_out, W,
                            jnp.reshape(b, (1, -1)), Ws, s)
        feats.append(f)

    sp = _sc_segment_sum_scalar(jnp.tile(s, (1, SW)), src, dst, zeros_sw)
    w_pad, sel_pad = _tc_select(
        jnp.reshape(sp[0, :, 0], (NROWS, 128)),
        jnp.reshape(sp[1, :, 0], (NROWS, 128)),
        ri_pad, jnp.reshape(score_b, (1, 1)))
    w_col = jnp.reshape(w_pad, (NP, 1))
    sel_col = jnp.reshape(sel_pad, (NP, 1))

    logits, feat = _tc_head(
        feats[0], feats[1], feats[2], w_col, sel_col,
        lin1_W, jnp.reshape(lin1_b, (1, -1)),
        lin2_W, jnp.reshape(lin2_b, (1, -1)),
        lin3_W, jnp.reshape(lin3_b, (1, -1)))
    return (logits, feat)


# speedup vs baseline: 1.3103x; 1.3103x over previous
import functools

import jax
import jax.numpy as jnp
from jax import lax
from jax.experimental import pallas as pl
from jax.experimental.pallas import tpu as pltpu
from jax.experimental.pallas import tpu_sc as plsc

N = 10000
E = 320000
D = 128
K = 5000
NP = 10240
NROWS = 80

NC = 2
NS = 16
NW = NC * NS
EPW = E // NW
CHUNK = 128
CH_PROC = 80
CH_ALL = 82
SLOTS = CH_ALL * CHUNK
RPS = NP // NS
SW = 8

_SC_MESH = dict(core_axis_name="c", subcore_axis_name="s")


def _sc_degrees(src, dst, ones_c, zeros_col):
    @functools.partial(
        pl.kernel,
        out_type=(
            jax.ShapeDtypeStruct((NC, NP, SW), jnp.float32),
            jax.ShapeDtypeStruct((NC, NP, SW), jnp.float32),
        ),
        mesh=plsc.VectorSubcoreMesh(**_SC_MESH),
        scratch_types=[
            pltpu.VMEM((CHUNK,), jnp.int32),
            pltpu.VMEM((CHUNK, SW), jnp.float32),
            pltpu.VMEM_SHARED((NP, SW), jnp.float32),
            pltpu.VMEM_SHARED((NP, SW), jnp.float32),
        ],
        compiler_params=pltpu.CompilerParams(use_tc_tiling_on_sc=False),
    )
    def body(src_hbm, dst_hbm, ones_hbm, zcol_hbm, dego_hbm, degi_hbm,
             idxv, onesv, acc_o, acc_i):
        cid = lax.axis_index("c")
        sid = lax.axis_index("s")
        wid = cid * NS + sid
        pltpu.sync_copy(ones_hbm, onesv)
        sl = pl.ds(sid * RPS, RPS)
        pltpu.sync_copy(zcol_hbm.at[sl], acc_o.at[sl])
        pltpu.sync_copy(zcol_hbm.at[sl], acc_i.at[sl])
        plsc.subcore_barrier()
        base0 = wid * SLOTS

        def step(i, carry):
            base = base0 + i * CHUNK
            pltpu.sync_copy(src_hbm.at[pl.ds(base, CHUNK)], idxv)
            pltpu.sync_copy(onesv, acc_o.at[idxv], add=True)
            pltpu.sync_copy(dst_hbm.at[pl.ds(base, CHUNK)], idxv)
            pltpu.sync_copy(onesv, acc_i.at[idxv], add=True)
            return carry

        lax.fori_loop(0, CH_ALL, step, 0)
        plsc.subcore_barrier()
        pltpu.sync_copy(acc_o.at[sl], dego_hbm.at[cid, sl])
        pltpu.sync_copy(acc_i.at[sl], degi_hbm.at[cid, sl])

    return body(src, dst, ones_c, zeros_col)


def _sc_segment_sum(h, src, dst, zeros):
    @functools.partial(
        pl.kernel,
        out_type=jax.ShapeDtypeStruct((NC, NP, D), jnp.float32),
        mesh=plsc.VectorSubcoreMesh(**_SC_MESH),
        scratch_types=[
            pltpu.VMEM((CHUNK,), jnp.int32),
            pltpu.VMEM((CHUNK,), jnp.int32),
            pltpu.VMEM((CHUNK,), jnp.int32),
            pltpu.VMEM((CHUNK,), jnp.int32),
            pltpu.VMEM((CHUNK, D), jnp.float32),
            pltpu.VMEM((CHUNK, D), jnp.float32),
            pltpu.VMEM_SHARED((NP, D), jnp.float32),
            pltpu.SemaphoreType.DMA,
            pltpu.SemaphoreType.DMA,
        ],
    )
    def body(h_hbm, src_hbm, dst_hbm, z_hbm, out_hbm,
             srcv0, dstv0, srcv1, dstv1, rows0, rows1, acc, sem0, sem1):
        cid = lax.axis_index("c")
        sid = lax.axis_index("s")
        wid = cid * NS + sid
        sl = pl.ds(sid * RPS, RPS)
        pltpu.sync_copy(z_hbm.at[sl], acc.at[sl])
        plsc.subcore_barrier()
        base0 = wid * SLOTS

        def fetch(c, srcv, dstv, rows, sem):
            base = base0 + c * CHUNK
            pltpu.sync_copy(src_hbm.at[pl.ds(base, CHUNK)], srcv)
            pltpu.sync_copy(dst_hbm.at[pl.ds(base, CHUNK)], dstv)
            pltpu.async_copy(h_hbm.at[srcv], rows, sem)

        def step(i, carry):
            base = base0 + i * CHUNK
            pltpu.sync_copy(src_hbm.at[pl.ds(base, CHUNK)], srcv0)
            pltpu.sync_copy(dst_hbm.at[pl.ds(base, CHUNK)], dstv0)
            pltpu.async_copy(h_hbm.at[srcv0], rows0, sem0).wait()
            pltpu.sync_copy(rows0, acc.at[dstv0], add=True)
            return carry

        lax.fori_loop(0, CH_PROC, step, 0)
        plsc.subcore_barrier()
        pltpu.sync_copy(acc.at[sl], out_hbm.at[cid, sl])

    return body(h, src, dst, zeros)


def _sc_segment_sum_scalar(s, src, dst, zeros_col):
    @functools.partial(
        pl.kernel,
        out_type=jax.ShapeDtypeStruct((NC, NP, SW), jnp.float32),
        mesh=plsc.VectorSubcoreMesh(**_SC_MESH),
        scratch_types=[
            pltpu.VMEM((CHUNK,), jnp.int32),
            pltpu.VMEM((CHUNK,), jnp.int32),
            pltpu.VMEM((CHUNK, SW), jnp.float32),
            pltpu.VMEM_SHARED((NP, SW), jnp.float32),
            pltpu.SemaphoreType.DMA,
        ],
        compiler_params=pltpu.CompilerParams(use_tc_tiling_on_sc=False),
    )
    def body(s_hbm, src_hbm, dst_hbm, z_hbm, out_hbm, srcv, dstv, vals, acc, sem):
        cid = lax.axis_index("c")
        sid = lax.axis_index("s")
        wid = cid * NS + sid
        sl = pl.ds(sid * RPS, RPS)
        pltpu.sync_copy(z_hbm.at[sl], acc.at[sl])
        plsc.subcore_barrier()
        base0 = wid * SLOTS

        def step(i, carry):
            base = base0 + i * CHUNK
            pltpu.sync_copy(src_hbm.at[pl.ds(base, CHUNK)], srcv)
            pltpu.sync_copy(dst_hbm.at[pl.ds(base, CHUNK)], dstv)
            pltpu.async_copy(s_hbm.at[srcv], vals, sem).wait()
            pltpu.sync_copy(vals, acc.at[dstv], add=True)
            return carry

        lax.fori_loop(0, CH_ALL, step, 0)
        plsc.subcore_barrier()
        pltpu.sync_copy(acc.at[sl], out_hbm.at[cid, sl])

    return body(s, src, dst, zeros_col)


def _tc_prep(do0, do1, di0, di1):
    def body(a_ref, b_ref, c_ref, d_ref, ro_ref, ri_ref):
        ro_ref[...] = lax.rsqrt(jnp.maximum(a_ref[...] + b_ref[...], 1.0))
        ri_ref[...] = lax.rsqrt(jnp.maximum(c_ref[...] + d_ref[...], 1.0))

    return pl.pallas_call(
        body,
        out_shape=(
            jax.ShapeDtypeStruct((NROWS, 128), jnp.float32),
            jax.ShapeDtypeStruct((NROWS, 128), jnp.float32),
        ),
    )(do0, do1, di0, di1)


_BLK = 512
_NBLK = NP // _BLK


def _tc_scale_rows(x, r_col):
    def body(x_ref, r_ref, o_ref):
        o_ref[...] = x_ref[...] * r_ref[...]

    return pl.pallas_call(
        body,
        grid=(_NBLK,),
        in_specs=[
            pl.BlockSpec((_BLK, D), lambda i: (i, 0)),
            pl.BlockSpec((_BLK, 1), lambda i: (i, 0)),
        ],
        out_specs=pl.BlockSpec((_BLK, D), lambda i: (i, 0)),
        out_shape=jax.ShapeDtypeStruct((NP, D), jnp.float32),
    )(x, r_col)


def _tc_layer(m0, m1, r_in, r_out, W, b, Ws, s_in):
    def body(m0_ref, m1_ref, ri_ref, ro_ref, w_ref, b_ref, ws_ref, si_ref,
             f_ref, h_ref, s_ref):
        m = (m0_ref[...] + m1_ref[...]) * ri_ref[...]
        f = jnp.dot(m, w_ref[...], preferred_element_type=jnp.float32) + b_ref[...]
        h = f * ro_ref[...]
        f_ref[...] = f
        h_ref[...] = h
        s_ref[...] = si_ref[...] + jnp.dot(h, ws_ref[...],
                                           preferred_element_type=jnp.float32)

    return pl.pallas_call(
        body,
        grid=(_NBLK,),
        in_specs=[
            pl.BlockSpec((_BLK, D), lambda i: (i, 0)),
            pl.BlockSpec((_BLK, D), lambda i: (i, 0)),
            pl.BlockSpec((_BLK, 1), lambda i: (i, 0)),
            pl.BlockSpec((_BLK, 1), lambda i: (i, 0)),
            pl.BlockSpec((D, D), lambda i: (0, 0)),
            pl.BlockSpec((1, D), lambda i: (0, 0)),
            pl.BlockSpec((D, 1), lambda i: (0, 0)),
            pl.BlockSpec((_BLK, 1), lambda i: (i, 0)),
        ],
        out_specs=(
            pl.BlockSpec((_BLK, D), lambda i: (i, 0)),
            pl.BlockSpec((_BLK, D), lambda i: (i, 0)),
            pl.BlockSpec((_BLK, 1), lambda i: (i, 0)),
        ),
        out_shape=(
            jax.ShapeDtypeStruct((NP, D), jnp.float32),
            jax.ShapeDtypeStruct((NP, D), jnp.float32),
            jax.ShapeDtypeStruct((NP, 1), jnp.float32),
        ),
    )(m0, m1, r_in, r_out, W, b, Ws, s_in)


def _tc_select(sp0, sp1, ri_pad, score_b):
    def body(a_ref, b_ref, r_ref, sb_ref, w_ref, sel_ref):
        score = (a_ref[...] + b_ref[...]) * r_ref[...] + sb_ref[0, 0]
        flat = (lax.broadcasted_iota(jnp.int32, (NROWS, 128), 0) * 128
                + lax.broadcasted_iota(jnp.int32, (NROWS, 128), 1))
        valid = flat < N
        score = jnp.where(valid, score, -jnp.inf)
        u = lax.bitcast_convert_type(score, jnp.uint32)
        key = jnp.where(u >> 31 == jnp.uint32(1), ~u, u | jnp.uint32(0x80000000))

        def bstep(_, lohi):
            lo, hi = lohi
            mid = lo + ((hi - lo) >> 1)
            c = jnp.sum((key > mid).astype(jnp.int32))
            pred = c < K
            return (jnp.where(pred, lo, mid + 1), jnp.where(pred, mid, hi))

        lo, _ = lax.fori_loop(0, 32, bstep,
                              (jnp.uint32(0), jnp.uint32(0xFFFFFFFF)))
        tau = lo
        gt = key > tau
        eq = key == tau
        c_gt = jnp.sum(gt.astype(jnp.int32))
        need = (K - c_gt).astype(jnp.float32)
        eqf = eq.astype(jnp.float32)
        cj = (lax.broadcasted_iota(jnp.int32, (128, 128), 0)
              < lax.broadcasted_iota(jnp.int32, (128, 128), 1))
        in_row = jnp.dot(eqf, cj.astype(jnp.float32),
                         preferred_element_type=jnp.float32)
        rows_eq = jnp.sum(eqf, axis=1, keepdims=True)
        rq = (lax.broadcasted_iota(jnp.int32, (NROWS, NROWS), 1)
              < lax.broadcasted_iota(jnp.int32, (NROWS, NROWS), 0))
        pre_row = jnp.dot(rq.astype(jnp.float32), rows_eq,
                          preferred_element_type=jnp.float32)
        rank = pre_row + in_row
        sel = gt | (eq & (rank < need))
        w_ref[...] = jnp.where(sel, jnp.tanh(score), 0.0)
        sel_ref[...] = sel.astype(jnp.float32)

    return pl.pallas_call(
        body,
        out_shape=(
            jax.ShapeDtypeStruct((NROWS, 128), jnp.float32),
            jax.ShapeDtypeStruct((NROWS, 128), jnp.float32),
        ),
    )(sp0, sp1, ri_pad, score_b)


def _tc_head(f1, f2, f3, w_col, sel_col, l1W, l1b, l2W, l2b, l3W, l3b):
    def body(f1_ref, f2_ref, f3_ref, w_ref, sel_ref,
             l1w_ref, l1b_ref, l2w_ref, l2b_ref, l3w_ref, l3b_ref,
             logits_ref, feat_ref):
        w = w_ref[...]
        selected = sel_ref[...] > 0.0
        parts_avg = []
        parts_max = []
        for f_ref in (f1_ref, f2_ref, f3_ref):
            p = f_ref[...] * w
            parts_avg.append(jnp.sum(p, axis=0, keepdims=True) * (1.0 / K))
            parts_max.append(jnp.max(jnp.where(selected, p, -3.4e38),
                                     axis=0, keepdims=True))
        feat0 = jnp.concatenate(parts_avg + parts_max, axis=1)
        h1 = jnp.maximum(
            jnp.dot(feat0, l1w_ref[...], preferred_element_type=jnp.float32)
            + l1b_ref[...], 0.0)
        h2 = jnp.maximum(
            jnp.dot(h1, l2w_ref[...], preferred_element_type=jnp.float32)
            + l2b_ref[...], 0.0)
        z = jnp.dot(h2, l3w_ref[...], preferred_element_type=jnp.float32) \
            + l3b_ref[...]
        zm = z - jnp.max(z, axis=1, keepdims=True)
        logits_ref[...] = zm - jnp.log(jnp.sum(jnp.exp(zm), axis=1,
                                               keepdims=True))
        feat_ref[...] = h2

    return pl.pallas_call(
        body,
        out_shape=(
            jax.ShapeDtypeStruct((1, 10), jnp.float32),
            jax.ShapeDtypeStruct((1, D), jnp.float32),
        ),
    )(f1, f2, f3, w_col, sel_col, l1W, l1b, l2W, l2b, l3W, l3b)


def kernel(x, edge_index, conv_W0, conv_b0, conv_W1, conv_b1, conv_W2,
           conv_b2, score_W, score_b, lin1_W, lin1_b, lin2_W, lin2_b,
           lin3_W, lin3_b):
    def pad_edges(a):
        a = jnp.reshape(a.astype(jnp.int32), (NW, EPW))
        a = jnp.pad(a, ((0, 0), (0, SLOTS - EPW)), constant_values=N)
        return jnp.reshape(a, (-1,))

    src = pad_edges(edge_index[0])
    dst = pad_edges(edge_index[1])
    x_pad = jnp.pad(x, ((0, NP - N), (0, 0)))
    zeros = jnp.zeros((NP, D), jnp.float32)
    zeros_col = jnp.zeros((NP, 1), jnp.float32)
    zeros_sw = jnp.zeros((NP, SW), jnp.float32)
    ones_c = jnp.ones((CHUNK, SW), jnp.float32)

    dego, degi = _sc_degrees(src, dst, ones_c, zeros_sw)
    ro_pad, ri_pad = _tc_prep(
        jnp.reshape(dego[0, :, 0], (NROWS, 128)),
        jnp.reshape(dego[1, :, 0], (NROWS, 128)),
        jnp.reshape(degi[0, :, 0], (NROWS, 128)),
        jnp.reshape(degi[1, :, 0], (NROWS, 128)))
    r_out = jnp.reshape(ro_pad, (NP, 1))
    r_in = jnp.reshape(ri_pad, (NP, 1))

    h = _tc_scale_rows(x_pad, r_out)
    s = zeros_col
    feats = []
    for W, b, Ws in (
            (conv_W0, conv_b0, score_W[0:D]),
            (conv_W1, conv_b1, score_W[D:2 * D]),
            (conv_W2, conv_b2, score_W[2 * D:3 * D])):
        m = _sc_segment_sum(h, src, dst, zeros)
        f, h, s = _tc_layer(m[0], m[1], r_in, r_out, W,
                            jnp.reshape(b, (1, -1)), Ws, s)
        feats.append(f)

    sp = _sc_segment_sum_scalar(jnp.tile(s, (1, SW)), src, dst, zeros_sw)
    w_pad, sel_pad = _tc_select(
        jnp.reshape(sp[0, :, 0], (NROWS, 128)),
        jnp.reshape(sp[1, :, 0], (NROWS, 128)),
        ri_pad, jnp.reshape(score_b, (1, 1)))
    w_col = jnp.reshape(w_pad, (NP, 1))
    sel_col = jnp.reshape(sel_pad, (NP, 1))

    logits, feat = _tc_head(
        feats[0], feats[1], feats[2], w_col, sel_col,
        lin1_W, jnp.reshape(lin1_b, (1, -1)),
        lin2_W, jnp.reshape(lin2_b, (1, -1)),
        lin3_W, jnp.reshape(lin3_b, (1, -1)))
    return (logits, feat)


# speedup vs baseline: 1.8378x; 1.4026x over previous
import functools

import jax
import jax.numpy as jnp
from jax import lax
from jax.experimental import pallas as pl
from jax.experimental.pallas import tpu as pltpu
from jax.experimental.pallas import tpu_sc as plsc

N = 10000
E = 320000
D = 128
K = 5000
NP = 10240
NROWS = 80

NC = 2
NS = 16
NW = NC * NS
EPW = E // NW
CHUNK = 80
CH_PROC = 125
CH_ALL = 127
SLOTS = CH_ALL * CHUNK
RPS = NP // NS
SW = 8

_SC_MESH = dict(core_axis_name="c", subcore_axis_name="s")


def _sc_degrees(src, dst, ones_c, zeros_col):
    @functools.partial(
        pl.kernel,
        out_type=(
            jax.ShapeDtypeStruct((NC, NP, SW), jnp.float32),
            jax.ShapeDtypeStruct((NC, NP, SW), jnp.float32),
        ),
        mesh=plsc.VectorSubcoreMesh(**_SC_MESH),
        scratch_types=[
            pltpu.VMEM((CHUNK,), jnp.int32),
            pltpu.VMEM((CHUNK, SW), jnp.float32),
            pltpu.VMEM_SHARED((NP, SW), jnp.float32),
            pltpu.VMEM_SHARED((NP, SW), jnp.float32),
        ],
        compiler_params=pltpu.CompilerParams(use_tc_tiling_on_sc=False),
    )
    def body(src_hbm, dst_hbm, ones_hbm, zcol_hbm, dego_hbm, degi_hbm,
             idxv, onesv, acc_o, acc_i):
        cid = lax.axis_index("c")
        sid = lax.axis_index("s")
        wid = cid * NS + sid
        pltpu.sync_copy(ones_hbm, onesv)
        sl = pl.ds(sid * RPS, RPS)
        pltpu.sync_copy(zcol_hbm.at[sl], acc_o.at[sl])
        pltpu.sync_copy(zcol_hbm.at[sl], acc_i.at[sl])
        plsc.subcore_barrier()
        base0 = wid * SLOTS

        def step(i, carry):
            base = base0 + i * CHUNK
            pltpu.sync_copy(src_hbm.at[pl.ds(base, CHUNK)], idxv)
            pltpu.sync_copy(onesv, acc_o.at[idxv], add=True)
            pltpu.sync_copy(dst_hbm.at[pl.ds(base, CHUNK)], idxv)
            pltpu.sync_copy(onesv, acc_i.at[idxv], add=True)
            return carry

        lax.fori_loop(0, CH_ALL, step, 0)
        plsc.subcore_barrier()
        pltpu.sync_copy(acc_o.at[sl], dego_hbm.at[cid, sl])
        pltpu.sync_copy(acc_i.at[sl], degi_hbm.at[cid, sl])

    return body(src, dst, ones_c, zeros_col)


def _sc_segment_sum(h, src, dst, zeros):
    @functools.partial(
        pl.kernel,
        out_type=jax.ShapeDtypeStruct((NC, NP, D), jnp.float32),
        mesh=plsc.VectorSubcoreMesh(**_SC_MESH),
        scratch_types=[
            pltpu.VMEM((CHUNK,), jnp.int32),
            pltpu.VMEM((CHUNK,), jnp.int32),
            pltpu.VMEM((CHUNK,), jnp.int32),
            pltpu.VMEM((CHUNK,), jnp.int32),
            pltpu.VMEM((CHUNK, D), jnp.float32),
            pltpu.VMEM((CHUNK, D), jnp.float32),
            pltpu.VMEM_SHARED((NP, D), jnp.float32),
            pltpu.SemaphoreType.DMA,
            pltpu.SemaphoreType.DMA,
        ],
    )
    def body(h_hbm, src_hbm, dst_hbm, z_hbm, out_hbm,
             srcv0, dstv0, srcv1, dstv1, rows0, rows1, acc, sem0, sem1):
        cid = lax.axis_index("c")
        sid = lax.axis_index("s")
        wid = cid * NS + sid
        sl = pl.ds(sid * RPS, RPS)
        pltpu.sync_copy(z_hbm.at[sl], acc.at[sl])
        plsc.subcore_barrier()
        base0 = wid * SLOTS

        def fetch(c, srcv, dstv, rows, sem):
            base = base0 + c * CHUNK
            pltpu.sync_copy(src_hbm.at[pl.ds(base, CHUNK)], srcv)
            pltpu.sync_copy(dst_hbm.at[pl.ds(base, CHUNK)], dstv)
            pltpu.async_copy(h_hbm.at[srcv], rows, sem)

        def step(i, carry):
            base = base0 + i * CHUNK
            pltpu.sync_copy(src_hbm.at[pl.ds(base, CHUNK)], srcv0)
            pltpu.sync_copy(dst_hbm.at[pl.ds(base, CHUNK)], dstv0)
            pltpu.async_copy(h_hbm.at[srcv0], rows0, sem0).wait()
            pltpu.sync_copy(rows0, acc.at[dstv0], add=True)
            return carry

        lax.fori_loop(0, CH_PROC, step, 0)
        plsc.subcore_barrier()
        pltpu.sync_copy(acc.at[sl], out_hbm.at[cid, sl])

    return body(h, src, dst, zeros)


def _sc_segment_sum_scalar(s, src, dst, zeros_col):
    @functools.partial(
        pl.kernel,
        out_type=jax.ShapeDtypeStruct((NC, NP, SW), jnp.float32),
        mesh=plsc.VectorSubcoreMesh(**_SC_MESH),
        scratch_types=[
            pltpu.VMEM((CHUNK,), jnp.int32),
            pltpu.VMEM((CHUNK,), jnp.int32),
            pltpu.VMEM((CHUNK, SW), jnp.float32),
            pltpu.VMEM_SHARED((NP, SW), jnp.float32),
            pltpu.SemaphoreType.DMA,
        ],
        compiler_params=pltpu.CompilerParams(use_tc_tiling_on_sc=False),
    )
    def body(s_hbm, src_hbm, dst_hbm, z_hbm, out_hbm, srcv, dstv, vals, acc, sem):
        cid = lax.axis_index("c")
        sid = lax.axis_index("s")
        wid = cid * NS + sid
        sl = pl.ds(sid * RPS, RPS)
        pltpu.sync_copy(z_hbm.at[sl], acc.at[sl])
        plsc.subcore_barrier()
        base0 = wid * SLOTS

        def step(i, carry):
            base = base0 + i * CHUNK
            pltpu.sync_copy(src_hbm.at[pl.ds(base, CHUNK)], srcv)
            pltpu.sync_copy(dst_hbm.at[pl.ds(base, CHUNK)], dstv)
            pltpu.async_copy(s_hbm.at[srcv], vals, sem).wait()
            pltpu.sync_copy(vals, acc.at[dstv], add=True)
            return carry

        lax.fori_loop(0, CH_ALL, step, 0)
        plsc.subcore_barrier()
        pltpu.sync_copy(acc.at[sl], out_hbm.at[cid, sl])

    return body(s, src, dst, zeros_col)


def _tc_prep(do0, do1, di0, di1):
    def body(a_ref, b_ref, c_ref, d_ref, ro_ref, ri_ref):
        ro_ref[...] = lax.rsqrt(jnp.maximum(a_ref[...] + b_ref[...], 1.0))
        ri_ref[...] = lax.rsqrt(jnp.maximum(c_ref[...] + d_ref[...], 1.0))

    return pl.pallas_call(
        body,
        out_shape=(
            jax.ShapeDtypeStruct((NROWS, 128), jnp.float32),
            jax.ShapeDtypeStruct((NROWS, 128), jnp.float32),
        ),
    )(do0, do1, di0, di1)


_BLK = 512
_NBLK = NP // _BLK


def _tc_scale_rows(x, r_col):
    def body(x_ref, r_ref, o_ref):
        o_ref[...] = x_ref[...] * r_ref[...]

    return pl.pallas_call(
        body,
        grid=(_NBLK,),
        in_specs=[
            pl.BlockSpec((_BLK, D), lambda i: (i, 0)),
            pl.BlockSpec((_BLK, 1), lambda i: (i, 0)),
        ],
        out_specs=pl.BlockSpec((_BLK, D), lambda i: (i, 0)),
        out_shape=jax.ShapeDtypeStruct((NP, D), jnp.float32),
    )(x, r_col)


def _tc_layer(m0, m1, r_in, r_out, W, b, Ws, s_in):
    def body(m0_ref, m1_ref, ri_ref, ro_ref, w_ref, b_ref, ws_ref, si_ref,
             f_ref, h_ref, s_ref):
        m = (m0_ref[...] + m1_ref[...]) * ri_ref[...]
        f = jnp.dot(m, w_ref[...], preferred_element_type=jnp.float32) + b_ref[...]
        h = f * ro_ref[...]
        f_ref[...] = f
        h_ref[...] = h
        s_ref[...] = si_ref[...] + jnp.dot(h, ws_ref[...],
                                           preferred_element_type=jnp.float32)

    return pl.pallas_call(
        body,
        grid=(_NBLK,),
        in_specs=[
            pl.BlockSpec((_BLK, D), lambda i: (i, 0)),
            pl.BlockSpec((_BLK, D), lambda i: (i, 0)),
            pl.BlockSpec((_BLK, 1), lambda i: (i, 0)),
            pl.BlockSpec((_BLK, 1), lambda i: (i, 0)),
            pl.BlockSpec((D, D), lambda i: (0, 0)),
            pl.BlockSpec((1, D), lambda i: (0, 0)),
            pl.BlockSpec((D, 1), lambda i: (0, 0)),
            pl.BlockSpec((_BLK, 1), lambda i: (i, 0)),
        ],
        out_specs=(
            pl.BlockSpec((_BLK, D), lambda i: (i, 0)),
            pl.BlockSpec((_BLK, D), lambda i: (i, 0)),
            pl.BlockSpec((_BLK, 1), lambda i: (i, 0)),
        ),
        out_shape=(
            jax.ShapeDtypeStruct((NP, D), jnp.float32),
            jax.ShapeDtypeStruct((NP, D), jnp.float32),
            jax.ShapeDtypeStruct((NP, 1), jnp.float32),
        ),
    )(m0, m1, r_in, r_out, W, b, Ws, s_in)


def _tc_select(sp0, sp1, ri_pad, score_b):
    def body(a_ref, b_ref, r_ref, sb_ref, w_ref, sel_ref):
        score = (a_ref[...] + b_ref[...]) * r_ref[...] + sb_ref[0, 0]
        flat = (lax.broadcasted_iota(jnp.int32, (NROWS, 128), 0) * 128
                + lax.broadcasted_iota(jnp.int32, (NROWS, 128), 1))
        valid = flat < N
        score = jnp.where(valid, score, -jnp.inf)
        u = lax.bitcast_convert_type(score, jnp.uint32)
        key = jnp.where(u >> 31 == jnp.uint32(1), ~u, u | jnp.uint32(0x80000000))

        def bstep(_, lohi):
            lo, hi = lohi
            mid = lo + ((hi - lo) >> 1)
            c = jnp.sum((key > mid).astype(jnp.int32))
            pred = c < K
            return (jnp.where(pred, lo, mid + 1), jnp.where(pred, mid, hi))

        lo, _ = lax.fori_loop(0, 32, bstep,
                              (jnp.uint32(0), jnp.uint32(0xFFFFFFFF)))
        tau = lo
        gt = key > tau
        eq = key == tau
        c_gt = jnp.sum(gt.astype(jnp.int32))
        need = (K - c_gt).astype(jnp.float32)
        eqf = eq.astype(jnp.float32)
        cj = (lax.broadcasted_iota(jnp.int32, (128, 128), 0)
              < lax.broadcasted_iota(jnp.int32, (128, 128), 1))
        in_row = jnp.dot(eqf, cj.astype(jnp.float32),
                         preferred_element_type=jnp.float32)
        rows_eq = jnp.sum(eqf, axis=1, keepdims=True)
        rq = (lax.broadcasted_iota(jnp.int32, (NROWS, NROWS), 1)
              < lax.broadcasted_iota(jnp.int32, (NROWS, NROWS), 0))
        pre_row = jnp.dot(rq.astype(jnp.float32), rows_eq,
                          preferred_element_type=jnp.float32)
        rank = pre_row + in_row
        sel = gt | (eq & (rank < need))
        w_ref[...] = jnp.where(sel, jnp.tanh(score), 0.0)
        sel_ref[...] = sel.astype(jnp.float32)

    return pl.pallas_call(
        body,
        out_shape=(
            jax.ShapeDtypeStruct((NROWS, 128), jnp.float32),
            jax.ShapeDtypeStruct((NROWS, 128), jnp.float32),
        ),
    )(sp0, sp1, ri_pad, score_b)


def _tc_head(f1, f2, f3, w_col, sel_col, l1W, l1b, l2W, l2b, l3W, l3b):
    def body(f1_ref, f2_ref, f3_ref, w_ref, sel_ref,
             l1w_ref, l1b_ref, l2w_ref, l2b_ref, l3w_ref, l3b_ref,
             logits_ref, feat_ref):
        w = w_ref[...]
        selected = sel_ref[...] > 0.0
        parts_avg = []
        parts_max = []
        for f_ref in (f1_ref, f2_ref, f3_ref):
            p = f_ref[...] * w
            parts_avg.append(jnp.sum(p, axis=0, keepdims=True) * (1.0 / K))
            parts_max.append(jnp.max(jnp.where(selected, p, -3.4e38),
                                     axis=0, keepdims=True))
        feat0 = jnp.concatenate(parts_avg + parts_max, axis=1)
        h1 = jnp.maximum(
            jnp.dot(feat0, l1w_ref[...], preferred_element_type=jnp.float32)
            + l1b_ref[...], 0.0)
        h2 = jnp.maximum(
            jnp.dot(h1, l2w_ref[...], preferred_element_type=jnp.float32)
            + l2b_ref[...], 0.0)
        z = jnp.dot(h2, l3w_ref[...], preferred_element_type=jnp.float32) \
            + l3b_ref[...]
        zm = z - jnp.max(z, axis=1, keepdims=True)
        logits_ref[...] = zm - jnp.log(jnp.sum(jnp.exp(zm), axis=1,
                                               keepdims=True))
        feat_ref[...] = h2

    return pl.pallas_call(
        body,
        out_shape=(
            jax.ShapeDtypeStruct((1, 10), jnp.float32),
            jax.ShapeDtypeStruct((1, D), jnp.float32),
        ),
    )(f1, f2, f3, w_col, sel_col, l1W, l1b, l2W, l2b, l3W, l3b)


def kernel(x, edge_index, conv_W0, conv_b0, conv_W1, conv_b1, conv_W2,
           conv_b2, score_W, score_b, lin1_W, lin1_b, lin2_W, lin2_b,
           lin3_W, lin3_b):
    def pad_edges(a):
        a = jnp.reshape(a.astype(jnp.int32), (NW, EPW))
        a = jnp.pad(a, ((0, 0), (0, SLOTS - EPW)), constant_values=N)
        return jnp.reshape(a, (-1,))

    src = pad_edges(edge_index[0])
    dst = pad_edges(edge_index[1])
    x_pad = jnp.pad(x, ((0, NP - N), (0, 0)))
    zeros = jnp.zeros((NP, D), jnp.float32)
    zeros_col = jnp.zeros((NP, 1), jnp.float32)
    zeros_sw = jnp.zeros((NP, SW), jnp.float32)
    ones_c = jnp.ones((CHUNK, SW), jnp.float32)

    dego, degi = _sc_degrees(src, dst, ones_c, zeros_sw)
    ro_pad, ri_pad = _tc_prep(
        jnp.reshape(dego[0, :, 0], (NROWS, 128)),
        jnp.reshape(dego[1, :, 0], (NROWS, 128)),
        jnp.reshape(degi[0, :, 0], (NROWS, 128)),
        jnp.reshape(degi[1, :, 0], (NROWS, 128)))
    r_out = jnp.reshape(ro_pad, (NP, 1))
    r_in = jnp.reshape(ri_pad, (NP, 1))

    h = _tc_scale_rows(x_pad, r_out)
    s = zeros_col
    feats = []
    for W, b, Ws in (
            (conv_W0, conv_b0, score_W[0:D]),
            (conv_W1, conv_b1, score_W[D:2 * D]),
            (conv_W2, conv_b2, score_W[2 * D:3 * D])):
        m = _sc_segment_sum(h, src, dst, zeros)
        f, h, s = _tc_layer(m[0], m[1], r_in, r_out, W,
                            jnp.reshape(b, (1, -1)), Ws, s)
        feats.append(f)

    sp = _sc_segment_sum_scalar(jnp.tile(s, (1, SW)), src, dst, zeros_sw)
    w_pad, sel_pad = _tc_select(
        jnp.reshape(sp[0, :, 0], (NROWS, 128)),
        jnp.reshape(sp[1, :, 0], (NROWS, 128)),
        ri_pad, jnp.reshape(score_b, (1, 1)))
    w_col = jnp.reshape(w_pad, (NP, 1))
    sel_col = jnp.reshape(sel_pad, (NP, 1))

    logits, feat = _tc_head(
        feats[0], feats[1], feats[2], w_col, sel_col,
        lin1_W, jnp.reshape(lin1_b, (1, -1)),
        lin2_W, jnp.reshape(lin2_b, (1, -1)),
        lin3_W, jnp.reshape(lin3_b, (1, -1)))
    return (logits, feat)


# speedup vs baseline: 2.8350x; 1.5426x over previous
import functools

import jax
import jax.numpy as jnp
from jax import lax
from jax.experimental import pallas as pl
from jax.experimental.pallas import tpu as pltpu
from jax.experimental.pallas import tpu_sc as plsc

N = 10000
E = 320000
D = 128
K = 5000
NP = 10240
NROWS = 80

NC = 2
NS = 16
NW = NC * NS
EPW = E // NW
CHUNK = 80
CH_ALL = 125
SLOTS = CH_ALL * CHUNK
RPS = NP // NS
SW = 8

_SC_MESH = dict(core_axis_name="c", subcore_axis_name="s")


def _sc_degrees(src, dst, ones_c, zeros_col):
    @functools.partial(
        pl.kernel,
        out_type=(
            jax.ShapeDtypeStruct((NC, NP, SW), jnp.float32),
            jax.ShapeDtypeStruct((NC, NP, SW), jnp.float32),
        ),
        mesh=plsc.VectorSubcoreMesh(**_SC_MESH),
        scratch_types=[
            pltpu.VMEM((CH_ALL, CHUNK), jnp.int32),
            pltpu.VMEM((CH_ALL, CHUNK), jnp.int32),
            pltpu.VMEM((CHUNK, SW), jnp.float32),
            pltpu.VMEM_SHARED((NP, SW), jnp.float32),
            pltpu.VMEM_SHARED((NP, SW), jnp.float32),
        ],
        compiler_params=pltpu.CompilerParams(use_tc_tiling_on_sc=False),
    )
    def body(src_hbm, dst_hbm, ones_hbm, zcol_hbm, dego_hbm, degi_hbm,
             srcv, dstv, onesv, acc_o, acc_i):
        cid = lax.axis_index("c")
        sid = lax.axis_index("s")
        wid = cid * NS + sid
        pltpu.sync_copy(ones_hbm, onesv)
        sl = pl.ds(sid * RPS, RPS)
        pltpu.sync_copy(zcol_hbm.at[sl], acc_o.at[sl])
        pltpu.sync_copy(zcol_hbm.at[sl], acc_i.at[sl])
        pltpu.sync_copy(src_hbm.at[wid], srcv)
        pltpu.sync_copy(dst_hbm.at[wid], dstv)
        plsc.subcore_barrier()

        def step(j, carry):
            pltpu.sync_copy(onesv, acc_o.at[srcv.at[j]], add=True)
            pltpu.sync_copy(onesv, acc_i.at[dstv.at[j]], add=True)
            return carry

        lax.fori_loop(0, CH_ALL, step, 0)
        plsc.subcore_barrier()
        pltpu.sync_copy(acc_o.at[sl], dego_hbm.at[cid, sl])
        pltpu.sync_copy(acc_i.at[sl], degi_hbm.at[cid, sl])

    return body(src, dst, ones_c, zeros_col)


def _sc_segment_sum(h, src, dst, zeros):
    @functools.partial(
        pl.kernel,
        out_type=jax.ShapeDtypeStruct((NC, NP, D), jnp.float32),
        mesh=plsc.VectorSubcoreMesh(**_SC_MESH),
        scratch_types=[
            pltpu.VMEM((CH_ALL, CHUNK), jnp.int32),
            pltpu.VMEM((CH_ALL, CHUNK), jnp.int32),
            pltpu.VMEM((CHUNK, D), jnp.float32),
            pltpu.VMEM_SHARED((NP, D), jnp.float32),
            pltpu.SemaphoreType.DMA,
        ],
    )
    def body(h_hbm, src_hbm, dst_hbm, z_hbm, out_hbm,
             srcv, dstv, rows, acc, sem):
        cid = lax.axis_index("c")
        sid = lax.axis_index("s")
        wid = cid * NS + sid
        sl = pl.ds(sid * RPS, RPS)
        pltpu.sync_copy(z_hbm.at[sl], acc.at[sl])
        pltpu.sync_copy(src_hbm.at[wid], srcv)
        pltpu.sync_copy(dst_hbm.at[wid], dstv)
        plsc.subcore_barrier()

        def step(j, carry):
            pltpu.async_copy(h_hbm.at[srcv.at[j]], rows, sem).wait()
            pltpu.sync_copy(rows, acc.at[dstv.at[j]], add=True)
            return carry

        lax.fori_loop(0, CH_ALL, step, 0)
        plsc.subcore_barrier()
        pltpu.sync_copy(acc.at[sl], out_hbm.at[cid, sl])

    return body(h, src, dst, zeros)


def _sc_segment_sum_scalar(s, src, dst, zeros_col):
    @functools.partial(
        pl.kernel,
        out_type=jax.ShapeDtypeStruct((NC, NP, SW), jnp.float32),
        mesh=plsc.VectorSubcoreMesh(**_SC_MESH),
        scratch_types=[
            pltpu.VMEM((CH_ALL, CHUNK), jnp.int32),
            pltpu.VMEM((CH_ALL, CHUNK), jnp.int32),
            pltpu.VMEM((CHUNK, SW), jnp.float32),
            pltpu.VMEM_SHARED((NP, SW), jnp.float32),
            pltpu.SemaphoreType.DMA,
        ],
        compiler_params=pltpu.CompilerParams(use_tc_tiling_on_sc=False),
    )
    def body(s_hbm, src_hbm, dst_hbm, z_hbm, out_hbm, srcv, dstv, vals, acc, sem):
        cid = lax.axis_index("c")
        sid = lax.axis_index("s")
        wid = cid * NS + sid
        sl = pl.ds(sid * RPS, RPS)
        pltpu.sync_copy(z_hbm.at[sl], acc.at[sl])
        pltpu.sync_copy(src_hbm.at[wid], srcv)
        pltpu.sync_copy(dst_hbm.at[wid], dstv)
        plsc.subcore_barrier()

        def step(j, carry):
            pltpu.async_copy(s_hbm.at[srcv.at[j]], vals, sem).wait()
            pltpu.sync_copy(vals, acc.at[dstv.at[j]], add=True)
            return carry

        lax.fori_loop(0, CH_ALL, step, 0)
        plsc.subcore_barrier()
        pltpu.sync_copy(acc.at[sl], out_hbm.at[cid, sl])

    return body(s, src, dst, zeros_col)


def _tc_prep(do0, do1, di0, di1):
    def body(a_ref, b_ref, c_ref, d_ref, ro_ref, ri_ref):
        ro_ref[...] = lax.rsqrt(jnp.maximum(a_ref[...] + b_ref[...], 1.0))
        ri_ref[...] = lax.rsqrt(jnp.maximum(c_ref[...] + d_ref[...], 1.0))

    return pl.pallas_call(
        body,
        out_shape=(
            jax.ShapeDtypeStruct((NROWS, 128), jnp.float32),
            jax.ShapeDtypeStruct((NROWS, 128), jnp.float32),
        ),
    )(do0, do1, di0, di1)


_BLK = 512
_NBLK = NP // _BLK


def _tc_scale_rows(x, r_col):
    def body(x_ref, r_ref, o_ref):
        o_ref[...] = x_ref[...] * r_ref[...]

    return pl.pallas_call(
        body,
        grid=(_NBLK,),
        in_specs=[
            pl.BlockSpec((_BLK, D), lambda i: (i, 0)),
            pl.BlockSpec((_BLK, 1), lambda i: (i, 0)),
        ],
        out_specs=pl.BlockSpec((_BLK, D), lambda i: (i, 0)),
        out_shape=jax.ShapeDtypeStruct((NP, D), jnp.float32),
    )(x, r_col)


def _tc_layer(m0, m1, r_in, r_out, W, b, Ws, s_in):
    def body(m0_ref, m1_ref, ri_ref, ro_ref, w_ref, b_ref, ws_ref, si_ref,
             f_ref, h_ref, s_ref):
        m = (m0_ref[...] + m1_ref[...]) * ri_ref[...]
        f = jnp.dot(m, w_ref[...], preferred_element_type=jnp.float32) + b_ref[...]
        h = f * ro_ref[...]
        f_ref[...] = f
        h_ref[...] = h
        s_ref[...] = si_ref[...] + jnp.dot(h, ws_ref[...],
                                           preferred_element_type=jnp.float32)

    return pl.pallas_call(
        body,
        grid=(_NBLK,),
        in_specs=[
            pl.BlockSpec((_BLK, D), lambda i: (i, 0)),
            pl.BlockSpec((_BLK, D), lambda i: (i, 0)),
            pl.BlockSpec((_BLK, 1), lambda i: (i, 0)),
            pl.BlockSpec((_BLK, 1), lambda i: (i, 0)),
            pl.BlockSpec((D, D), lambda i: (0, 0)),
            pl.BlockSpec((1, D), lambda i: (0, 0)),
            pl.BlockSpec((D, 1), lambda i: (0, 0)),
            pl.BlockSpec((_BLK, 1), lambda i: (i, 0)),
        ],
        out_specs=(
            pl.BlockSpec((_BLK, D), lambda i: (i, 0)),
            pl.BlockSpec((_BLK, D), lambda i: (i, 0)),
            pl.BlockSpec((_BLK, 1), lambda i: (i, 0)),
        ),
        out_shape=(
            jax.ShapeDtypeStruct((NP, D), jnp.float32),
            jax.ShapeDtypeStruct((NP, D), jnp.float32),
            jax.ShapeDtypeStruct((NP, 1), jnp.float32),
        ),
    )(m0, m1, r_in, r_out, W, b, Ws, s_in)


def _tc_select(sp0, sp1, ri_pad, score_b):
    def body(a_ref, b_ref, r_ref, sb_ref, w_ref, sel_ref):
        score = (a_ref[...] + b_ref[...]) * r_ref[...] + sb_ref[0, 0]
        flat = (lax.broadcasted_iota(jnp.int32, (NROWS, 128), 0) * 128
                + lax.broadcasted_iota(jnp.int32, (NROWS, 128), 1))
        valid = flat < N
        score = jnp.where(valid, score, -jnp.inf)
        u = lax.bitcast_convert_type(score, jnp.uint32)
        key = jnp.where(u >> 31 == jnp.uint32(1), ~u, u | jnp.uint32(0x80000000))

        def bstep(_, lohi):
            lo, hi = lohi
            mid = lo + ((hi - lo) >> 1)
            c = jnp.sum((key > mid).astype(jnp.int32))
            pred = c < K
            return (jnp.where(pred, lo, mid + 1), jnp.where(pred, mid, hi))

        lo, _ = lax.fori_loop(0, 32, bstep,
                              (jnp.uint32(0), jnp.uint32(0xFFFFFFFF)))
        tau = lo
        gt = key > tau
        eq = key == tau
        c_gt = jnp.sum(gt.astype(jnp.int32))
        need = (K - c_gt).astype(jnp.float32)
        eqf = eq.astype(jnp.float32)
        cj = (lax.broadcasted_iota(jnp.int32, (128, 128), 0)
              < lax.broadcasted_iota(jnp.int32, (128, 128), 1))
        in_row = jnp.dot(eqf, cj.astype(jnp.float32),
                         preferred_element_type=jnp.float32)
        rows_eq = jnp.sum(eqf, axis=1, keepdims=True)
        rq = (lax.broadcasted_iota(jnp.int32, (NROWS, NROWS), 1)
              < lax.broadcasted_iota(jnp.int32, (NROWS, NROWS), 0))
        pre_row = jnp.dot(rq.astype(jnp.float32), rows_eq,
                          preferred_element_type=jnp.float32)
        rank = pre_row + in_row
        sel = gt | (eq & (rank < need))
        w_ref[...] = jnp.where(sel, jnp.tanh(score), 0.0)
        sel_ref[...] = sel.astype(jnp.float32)

    return pl.pallas_call(
        body,
        out_shape=(
            jax.ShapeDtypeStruct((NROWS, 128), jnp.float32),
            jax.ShapeDtypeStruct((NROWS, 128), jnp.float32),
        ),
    )(sp0, sp1, ri_pad, score_b)


def _tc_head(f1, f2, f3, w_col, sel_col, l1W, l1b, l2W, l2b, l3W, l3b):
    def body(f1_ref, f2_ref, f3_ref, w_ref, sel_ref,
             l1w_ref, l1b_ref, l2w_ref, l2b_ref, l3w_ref, l3b_ref,
             logits_ref, feat_ref):
        w = w_ref[...]
        selected = sel_ref[...] > 0.0
        parts_avg = []
        parts_max = []
        for f_ref in (f1_ref, f2_ref, f3_ref):
            p = f_ref[...] * w
            parts_avg.append(jnp.sum(p, axis=0, keepdims=True) * (1.0 / K))
            parts_max.append(jnp.max(jnp.where(selected, p, -3.4e38),
                                     axis=0, keepdims=True))
        feat0 = jnp.concatenate(parts_avg + parts_max, axis=1)
        h1 = jnp.maximum(
            jnp.dot(feat0, l1w_ref[...], preferred_element_type=jnp.float32)
            + l1b_ref[...], 0.0)
        h2 = jnp.maximum(
            jnp.dot(h1, l2w_ref[...], preferred_element_type=jnp.float32)
            + l2b_ref[...], 0.0)
        z = jnp.dot(h2, l3w_ref[...], preferred_element_type=jnp.float32) \
            + l3b_ref[...]
        zm = z - jnp.max(z, axis=1, keepdims=True)
        logits_ref[...] = zm - jnp.log(jnp.sum(jnp.exp(zm), axis=1,
                                               keepdims=True))
        feat_ref[...] = h2

    return pl.pallas_call(
        body,
        out_shape=(
            jax.ShapeDtypeStruct((1, 10), jnp.float32),
            jax.ShapeDtypeStruct((1, D), jnp.float32),
        ),
    )(f1, f2, f3, w_col, sel_col, l1W, l1b, l2W, l2b, l3W, l3b)


def kernel(x, edge_index, conv_W0, conv_b0, conv_W1, conv_b1, conv_W2,
           conv_b2, score_W, score_b, lin1_W, lin1_b, lin2_W, lin2_b,
           lin3_W, lin3_b):
    src = jnp.reshape(edge_index[0].astype(jnp.int32), (NW, CH_ALL, CHUNK))
    dst = jnp.reshape(edge_index[1].astype(jnp.int32), (NW, CH_ALL, CHUNK))
    x_pad = jnp.pad(x, ((0, NP - N), (0, 0)))
    zeros = jnp.zeros((NP, D), jnp.float32)
    zeros_col = jnp.zeros((NP, 1), jnp.float32)
    zeros_sw = jnp.zeros((NP, SW), jnp.float32)
    ones_c = jnp.ones((CHUNK, SW), jnp.float32)

    dego, degi = _sc_degrees(src, dst, ones_c, zeros_sw)
    ro_pad, ri_pad = _tc_prep(
        jnp.reshape(dego[0, :, 0], (NROWS, 128)),
        jnp.reshape(dego[1, :, 0], (NROWS, 128)),
        jnp.reshape(degi[0, :, 0], (NROWS, 128)),
        jnp.reshape(degi[1, :, 0], (NROWS, 128)))
    r_out = jnp.reshape(ro_pad, (NP, 1))
    r_in = jnp.reshape(ri_pad, (NP, 1))

    h = _tc_scale_rows(x_pad, r_out)
    s = zeros_col
    feats = []
    for W, b, Ws in (
            (conv_W0, conv_b0, score_W[0:D]),
            (conv_W1, conv_b1, score_W[D:2 * D]),
            (conv_W2, conv_b2, score_W[2 * D:3 * D])):
        m = _sc_segment_sum(h, src, dst, zeros)
        f, h, s = _tc_layer(m[0], m[1], r_in, r_out, W,
                            jnp.reshape(b, (1, -1)), Ws, s)
        feats.append(f)

    sp = _sc_segment_sum_scalar(jnp.tile(s, (1, SW)), src, dst, zeros_sw)
    w_pad, sel_pad = _tc_select(
        jnp.reshape(sp[0, :, 0], (NROWS, 128)),
        jnp.reshape(sp[1, :, 0], (NROWS, 128)),
        ri_pad, jnp.reshape(score_b, (1, 1)))
    w_col = jnp.reshape(w_pad, (NP, 1))
    sel_col = jnp.reshape(sel_pad, (NP, 1))

    logits, feat = _tc_head(
        feats[0], feats[1], feats[2], w_col, sel_col,
        lin1_W, jnp.reshape(lin1_b, (1, -1)),
        lin2_W, jnp.reshape(lin2_b, (1, -1)),
        lin3_W, jnp.reshape(lin3_b, (1, -1)))
    return (logits, feat)
